# baseline (speedup 1.0000x reference)
import jax
import jax.numpy as jnp
from jax import lax
from jax.experimental import pallas as pl
from jax.experimental.pallas import tpu as pltpu
from jax.experimental.pallas import tpu_sc as plsc

N = 10000
D = 128
E_ER = 320000
E_EE = 160000
E_RR = 160000

NC, NS = 2, 16
NW = NC * NS
L = 16

K2_EPT = E_ER // NW
K2_K = 80
K2_NB = K2_EPT // K2_K

K3A_EPT = E_ER // NS
K3A_K = 80
K3A_NB = K3A_EPT // K3A_K
K3A_CH = 25
K3A_NCH = K3A_NB // K3A_CH

K3B_EPT = E_EE // NS
K3B_K = 80
K3B_NB = K3B_EPT // K3B_K

ROWS_PT = N // NS

F32 = jnp.float32
I32 = jnp.int32


def _k1_body(ne_ref, ael_ref, aeh_ref, arh_ref, arr_ref, be_ref, br_ref,
             pd_ref, ps_ref):
    x = ne_ref[...]
    bf = jnp.bfloat16
    pd_ref[:, :D] = (x @ ael_ref[...] + be_ref[...]).astype(bf)
    pd_ref[:, D:] = (x @ arr_ref[...]).astype(bf)
    ps_ref[:, :D] = (x @ aeh_ref[...]).astype(bf)
    ps_ref[:, D:] = (x @ arh_ref[...] + br_ref[...]).astype(bf)


def _k2_body(pd_hbm, ps_hbm, sidx_hbm, didx_hbm, w0_hbm, w1_hbm,
             we_hbm, wr_hbm, sse_hbm, ssr_hbm,
             sidx_v, didx_v, pd_rows, ps_rows, we_blk, wr_blk,
             sse_v, ssr_v, w0_v, w1_v, sem_pd, sem_ps, sem_we, sem_wr):
    wid = lax.axis_index("s") * NC + lax.axis_index("c")
    pltpu.sync_copy(sidx_hbm.at[wid], sidx_v)
    pltpu.sync_copy(didx_hbm.at[wid], didx_v)
    pltpu.sync_copy(w0_hbm, w0_v)
    pltpu.sync_copy(w1_hbm, w1_v)

    zeros16 = jnp.zeros((L,), F32)

    def zero_seg(i, _):
        sse_v[pl.ds(i * L, L)] = zeros16
        ssr_v[pl.ds(i * L, L)] = zeros16
        return 0

    lax.fori_loop(0, N // L, zero_seg, 0)

    lanes = lax.iota(I32, L)

    def issue(b, slot):
        pltpu.async_copy(pd_hbm.at[didx_v.at[b]], pd_rows.at[slot],
                         sem_pd.at[slot])
        pltpu.async_copy(ps_hbm.at[sidx_v.at[b]], ps_rows.at[slot],
                         sem_ps.at[slot])

    issue(0, 0)

    def block(b, _):
        slot = b & 1
        pltpu.make_async_copy(pd_hbm.at[pl.ds(0, K2_K)], pd_rows.at[slot],
                              sem_pd.at[slot]).wait()
        pltpu.make_async_copy(ps_hbm.at[pl.ds(0, K2_K)], ps_rows.at[slot],
                              sem_ps.at[slot]).wait()

        @pl.when(b + 1 < K2_NB)
        def _():
            issue(b + 1, 1 - slot)

        @pl.when(b >= 2)
        def _():
            pltpu.make_async_copy(
                we_blk.at[pl.ds(slot * K2_K, K2_K)],
                we_hbm.at[wid, pl.ds(0, K2_K)], sem_we.at[slot]).wait()
            pltpu.make_async_copy(
                wr_blk.at[pl.ds(slot * K2_K, K2_K)],
                wr_hbm.at[wid, pl.ds(0, K2_K)], sem_wr.at[slot]).wait()

        bufsplat = jnp.full((L,), slot, I32)
        for g in range(K2_K // L):
            rows16 = jnp.full((L,), g * L, I32) + lanes

            def unpk(x):
                return plsc.unpack(plsc.bitcast(x, jnp.bfloat16),
                                   format=plsc.PackFormat.INTERLEAVED,
                                   preferred_element_type=F32)

            def feat8(fi, accs):
                acc_e, acc_r = accs
                for j in range(8):
                    colw = (jnp.full((L,), fi * 8 + j, I32) + lanes) & (
                        D // 2 - 1)
                    pa0, pa1 = unpk(
                        plsc.load_gather(pd_rows, [bufsplat, rows16, colw]))
                    sa0, sa1 = unpk(
                        plsc.load_gather(ps_rows, [bufsplat, rows16, colw]))
                    wa0, wa1 = unpk(plsc.load_gather(w0_v, [colw]))
                    m0 = jnp.exp((pa0 + sa0) * -2.0)
                    m1 = jnp.exp((pa1 + sa1) * -2.0)
                    acc_e = (acc_e + wa0 * ((1.0 - m0) / (1.0 + m0))
                             + wa1 * ((1.0 - m1) / (1.0 + m1)))
                    colb = colw + D // 2
                    pb0, pb1 = unpk(
                        plsc.load_gather(pd_rows, [bufsplat, rows16, colb]))
                    sb0, sb1 = unpk(
                        plsc.load_gather(ps_rows, [bufsplat, rows16, colb]))
                    wb0, wb1 = unpk(plsc.load_gather(w1_v, [colw]))
                    n0 = jnp.exp((pb0 + sb0) * -2.0)
                    n1 = jnp.exp((pb1 + sb1) * -2.0)
                    acc_r = (acc_r + wb0 * ((1.0 - n0) / (1.0 + n0))
                             + wb1 * ((1.0 - n1) / (1.0 + n1)))
                return acc_e, acc_r

            acc_e, acc_r = lax.fori_loop(0, D // 16, feat8,
                                         (zeros16, zeros16))
            we16 = jnp.exp(acc_e)
            wr16 = jnp.exp(acc_r)
            off = slot * K2_K + g * L
            we_blk[pl.ds(off, L)] = we16
            wr_blk[pl.ds(off, L)] = wr16
            bsplat = jnp.full((L,), b, I32)
            s16 = plsc.load_gather(sidx_v, [bsplat, rows16])
            d16 = plsc.load_gather(didx_v, [bsplat, rows16])
            plsc.addupdate_scatter(sse_v, [s16], we16)
            plsc.addupdate_scatter(ssr_v, [d16], wr16)
        pltpu.async_copy(we_blk.at[pl.ds(slot * K2_K, K2_K)],
                         we_hbm.at[wid, pl.ds(b * K2_K, K2_K)],
                         sem_we.at[slot])
        pltpu.async_copy(wr_blk.at[pl.ds(slot * K2_K, K2_K)],
                         wr_hbm.at[wid, pl.ds(b * K2_K, K2_K)],
                         sem_wr.at[slot])
        return 0

    lax.fori_loop(0, K2_NB, block, 0)

    for slot in (0, 1):
        pltpu.make_async_copy(we_blk.at[pl.ds(slot * K2_K, K2_K)],
                              we_hbm.at[wid, pl.ds(0, K2_K)],
                              sem_we.at[slot]).wait()
        pltpu.make_async_copy(wr_blk.at[pl.ds(slot * K2_K, K2_K)],
                              wr_hbm.at[wid, pl.ds(0, K2_K)],
                              sem_wr.at[slot]).wait()
    pltpu.sync_copy(sse_v, sse_hbm.at[wid])
    pltpu.sync_copy(ssr_v, ssr_hbm.at[wid])


def _k3a_body(node_hbm, gidx_hbm, sidx_hbm, w_hbm, zeros_hbm,
              agg_hbm,
              gidx_v, sidx_v, w_v, rows_v, acc_sh, sem_a):
    cid = lax.axis_index("c")
    sid = lax.axis_index("s")
    pltpu.sync_copy(zeros_hbm, acc_sh.at[pl.ds(sid * ROWS_PT, ROWS_PT)])
    plsc.subcore_barrier()

    def chunk(c, _):
        pltpu.sync_copy(gidx_hbm.at[cid, sid, pl.ds(c * K3A_CH, K3A_CH)],
                        gidx_v)
        pltpu.sync_copy(sidx_hbm.at[cid, sid, pl.ds(c * K3A_CH, K3A_CH)],
                        sidx_v)
        pltpu.sync_copy(w_hbm.at[cid, sid, pl.ds(c * K3A_CH * K3A_K,
                                                 K3A_CH * K3A_K)], w_v)
        pltpu.async_copy(node_hbm.at[gidx_v.at[0]], rows_v.at[0],
                         sem_a.at[0])

        def block(b, _):
            slot = b & 1
            pltpu.make_async_copy(node_hbm.at[pl.ds(0, K3A_K)],
                                  rows_v.at[slot], sem_a.at[slot]).wait()

            @pl.when(b + 1 < K3A_CH)
            def _():
                pltpu.async_copy(node_hbm.at[gidx_v.at[b + 1]],
                                 rows_v.at[1 - slot], sem_a.at[1 - slot])

            rows = rows_v.at[slot]
            for e in range(K3A_K):
                wb = plsc.load_gather(
                    w_v, [jnp.full((L,), b * K3A_K + e, I32)])
                for c in range(D // L):
                    rows[e, c * L:(c + 1) * L] = (
                        rows[e, c * L:(c + 1) * L] * wb)
            pltpu.sync_copy(rows, acc_sh.at[sidx_v.at[b]], add=True)
            return 0

        lax.fori_loop(0, K3A_CH, block, 0)
        return 0

    lax.fori_loop(0, K3A_NCH, chunk, 0)
    plsc.subcore_barrier()
    pltpu.sync_copy(acc_sh.at[pl.ds(sid * ROWS_PT, ROWS_PT)],
                    agg_hbm.at[cid, sid])


def _k3b_body(node_hbm, gidx_hbm, sidx_hbm, zeros_hbm,
              msum_hbm, cnt_hbm,
              gidx_v, sidx_v, rows_v, cnt_v, acc_sh, sem_a):
    cid = lax.axis_index("c")
    sid = lax.axis_index("s")
    pltpu.sync_copy(gidx_hbm.at[cid, sid], gidx_v)
    pltpu.sync_copy(sidx_hbm.at[cid, sid], sidx_v)
    pltpu.sync_copy(zeros_hbm, acc_sh.at[pl.ds(sid * ROWS_PT, ROWS_PT)])

    zeros16 = jnp.zeros((L,), F32)

    def zero_cnt(i, _):
        cnt_v[pl.ds(i * L, L)] = zeros16
        return 0

    lax.fori_loop(0, N // L, zero_cnt, 0)
    plsc.subcore_barrier()

    lanes = lax.iota(I32, L)
    ones16 = jnp.ones((L,), F32)

    pltpu.async_copy(node_hbm.at[gidx_v.at[0]], rows_v.at[0], sem_a.at[0])

    def block(b, _):
        slot = b & 1
        pltpu.make_async_copy(node_hbm.at[pl.ds(0, K3B_K)],
                              rows_v.at[slot], sem_a.at[slot]).wait()

        @pl.when(b + 1 < K3B_NB)
        def _():
            pltpu.async_copy(node_hbm.at[gidx_v.at[b + 1]],
                             rows_v.at[1 - slot], sem_a.at[1 - slot])

        bsplat = jnp.full((L,), b, I32)
        for g in range(K3B_K // L):
            cols = jnp.full((L,), g * L, I32) + lanes
            s16 = plsc.load_gather(sidx_v, [bsplat, cols])
            plsc.addupdate_scatter(cnt_v, [s16], ones16)
        pltpu.sync_copy(rows_v.at[slot], acc_sh.at[sidx_v.at[b]], add=True)
        return 0

    lax.fori_loop(0, K3B_NB, block, 0)
    plsc.subcore_barrier()
    pltpu.sync_copy(acc_sh.at[pl.ds(sid * ROWS_PT, ROWS_PT)],
                    msum_hbm.at[cid, sid])
    pltpu.sync_copy(cnt_v, cnt_hbm.at[cid, sid])


def _k4_body(ne_ref, agge_ref, aggr_ref, m0_ref, m1_ref, sse_ref, ssr_ref,
             cnt_ref, w1_ref, b1_ref, w2_ref, b2_ref, w3_ref, b3_ref,
             out_ref):
    inv_e = 1.0 / (jnp.sum(sse_ref[...], axis=1) + 1e-9)
    inv_r = 1.0 / (jnp.sum(ssr_ref[...], axis=1) + 1e-9)
    attn = agge_ref[...] * inv_e[:, None] + aggr_ref[...] * inv_r[:, None]
    cnt = jnp.maximum(jnp.sum(cnt_ref[...], axis=1), 1.0)
    mean = (m0_ref[...] + m1_ref[...]) / cnt[:, None]
    out_ref[...] = (jnp.tanh(ne_ref[...] @ w1_ref[...] + b1_ref[...])
                    + jnp.tanh(attn @ w2_ref[...] + b2_ref[...])
                    + jnp.tanh(mean @ w3_ref[...] + b3_ref[...]))


def _sc_mesh():
    return plsc.VectorSubcoreMesh(core_axis_name="c", subcore_axis_name="s",
                                  num_cores=NC, num_subcores=NS)


_SC_PARAMS = pltpu.CompilerParams(use_tc_tiling_on_sc=False,
                                  needs_layout_passes=False)


def kernel(node_emb, er_src, er_dst, ee_src, ee_dst, rr_src, rr_dst,
           W_attn_e_w, W_attn_e_b, w0_w, w0_b,
           W_attn_r_w, W_attn_r_b, w1_w, w1_b,
           W1_w, W1_b, W2_w, W2_b, W3_w, W3_b):
    er_src = er_src.astype(I32)
    er_dst = er_dst.astype(I32)
    ee_src = ee_src.astype(I32)
    ee_dst = ee_dst.astype(I32)
    rr_src = rr_src.astype(I32)
    rr_dst = rr_dst.astype(I32)

    n_blk = 1000
    k1_blk = 2000
    row_spec = pl.BlockSpec((n_blk, D), lambda i: (i, 0))
    w_spec = pl.BlockSpec((D, D), lambda i: (0, 0))
    b_spec = pl.BlockSpec((D,), lambda i: (0,))
    k1_row_spec = pl.BlockSpec((k1_blk, D), lambda i: (i, 0))
    k1_wide_spec = pl.BlockSpec((k1_blk, 2 * D), lambda i: (i, 0))
    pd, ps = pl.pallas_call(
        _k1_body,
        grid=(N // k1_blk,),
        in_specs=[k1_row_spec, w_spec, w_spec, w_spec, w_spec,
                  b_spec, b_spec],
        out_specs=[k1_wide_spec, k1_wide_spec],
        out_shape=[jax.ShapeDtypeStruct((N, 2 * D), jnp.bfloat16),
                   jax.ShapeDtypeStruct((N, 2 * D), jnp.bfloat16)],
    )(node_emb,
      W_attn_e_w[:, :D].T, W_attn_e_w[:, D:].T,
      W_attn_r_w[:, :D].T, W_attn_r_w[:, D:].T,
      W_attn_e_b, W_attn_r_b)
    pd = lax.bitcast_convert_type(pd.reshape(N, D, 2), I32)
    ps = lax.bitcast_convert_type(ps.reshape(N, D, 2), I32)
    w0p = lax.bitcast_convert_type(
        w0_w[0].astype(jnp.bfloat16).reshape(D // 2, 2), I32)
    w1p = lax.bitcast_convert_type(
        w1_w[0].astype(jnp.bfloat16).reshape(D // 2, 2), I32)

    k2 = pl.kernel(
        _k2_body,
        out_type=[jax.ShapeDtypeStruct((NW, K2_EPT), F32),
                  jax.ShapeDtypeStruct((NW, K2_EPT), F32),
                  jax.ShapeDtypeStruct((NW, N), F32),
                  jax.ShapeDtypeStruct((NW, N), F32)],
        mesh=_sc_mesh(),
        compiler_params=_SC_PARAMS,
        scratch_types=[pltpu.VMEM((K2_NB, K2_K), I32),
                       pltpu.VMEM((K2_NB, K2_K), I32),
                       pltpu.VMEM((2, K2_K, D), I32),
                       pltpu.VMEM((2, K2_K, D), I32),
                       pltpu.VMEM((2 * K2_K,), F32),
                       pltpu.VMEM((2 * K2_K,), F32),
                       pltpu.VMEM((N,), F32),
                       pltpu.VMEM((N,), F32),
                       pltpu.VMEM((D // 2,), I32),
                       pltpu.VMEM((D // 2,), I32),
                       pltpu.SemaphoreType.DMA((2,)),
                       pltpu.SemaphoreType.DMA((2,)),
                       pltpu.SemaphoreType.DMA((2,)),
                       pltpu.SemaphoreType.DMA((2,))],
    )
    we, wr, sse_p, ssr_p = k2(pd, ps,
                              er_src.reshape(NW, K2_NB, K2_K),
                              er_dst.reshape(NW, K2_NB, K2_K),
                              w0p, w1p)

    zeros_init = jnp.zeros((ROWS_PT, D), F32)

    k3a = pl.kernel(
        _k3a_body,
        out_type=[jax.ShapeDtypeStruct((NC, NS, ROWS_PT, D), F32)],
        mesh=_sc_mesh(),
        compiler_params=_SC_PARAMS,
        scratch_types=[pltpu.VMEM((K3A_CH, K3A_K), I32),
                       pltpu.VMEM((K3A_CH, K3A_K), I32),
                       pltpu.VMEM((K3A_CH * K3A_K,), F32),
                       pltpu.VMEM((2, K3A_K, D), F32),
                       pltpu.VMEM_SHARED((N, D), F32),
                       pltpu.SemaphoreType.DMA((2,))],
    )
    gidx_a = jnp.stack([er_dst, er_src]).reshape(NC, NS, K3A_NB, K3A_K)
    sidx_a = jnp.stack([er_src, er_dst]).reshape(NC, NS, K3A_NB, K3A_K)
    w_all = jnp.stack([we.reshape(-1), wr.reshape(-1)]).reshape(
        NC, NS, K3A_EPT)
    (agg,) = k3a(node_emb, gidx_a, sidx_a, w_all, zeros_init)

    k3b = pl.kernel(
        _k3b_body,
        out_type=[jax.ShapeDtypeStruct((NC, NS, ROWS_PT, D), F32),
                  jax.ShapeDtypeStruct((NC, NS, N), F32)],
        mesh=_sc_mesh(),
        compiler_params=_SC_PARAMS,
        scratch_types=[pltpu.VMEM((K3B_NB, K3B_K), I32),
                       pltpu.VMEM((K3B_NB, K3B_K), I32),
                       pltpu.VMEM((2, K3B_K, D), F32),
                       pltpu.VMEM((N,), F32),
                       pltpu.VMEM_SHARED((N, D), F32),
                       pltpu.SemaphoreType.DMA((2,))],
    )
    gidx_b = jnp.stack([ee_dst, rr_dst]).reshape(NC, NS, K3B_NB, K3B_K)
    sidx_b = jnp.stack([ee_src, rr_src]).reshape(NC, NS, K3B_NB, K3B_K)
    msum, cnt_p = k3b(node_emb, gidx_b, sidx_b, zeros_init)

    agg = agg.reshape(NC, N, D)
    msum = msum.reshape(NC, N, D)
    part_spec = pl.BlockSpec((n_blk, NW), lambda i: (i, 0))
    out = pl.pallas_call(
        _k4_body,
        grid=(N // n_blk,),
        in_specs=[row_spec, row_spec, row_spec, row_spec, row_spec,
                  part_spec, part_spec, part_spec,
                  w_spec, b_spec, w_spec, b_spec, w_spec, b_spec],
        out_specs=row_spec,
        out_shape=jax.ShapeDtypeStruct((N, D), F32),
    )(node_emb, agg[0], agg[1], msum[0], msum[1],
      sse_p.T, ssr_p.T, cnt_p.reshape(NW, N).T,
      W1_w.T, W1_b, W2_w.T, W2_b, W3_w.T, W3_b)
    return out


# speedup vs baseline: 1.4415x; 1.4415x over previous
import jax
import jax.numpy as jnp
from jax import lax
from jax.experimental import pallas as pl
from jax.experimental.pallas import tpu as pltpu
from jax.experimental.pallas import tpu_sc as plsc

N = 10000
D = 128
E_ER = 320000
E_EE = 160000
E_RR = 160000

NC, NS = 2, 16
NW = NC * NS
L = 16

K2_EPT = E_ER // NW
K2_K = 80
K2_NB = K2_EPT // K2_K

K3A_EPT = E_ER // NS
K3A_K = 80
K3A_NB = K3A_EPT // K3A_K
K3A_CH = 25
K3A_NCH = K3A_NB // K3A_CH

K3B_EPT = E_EE // NS
K3B_K = 80
K3B_NB = K3B_EPT // K3B_K

ROWS_PT = N // NS

F32 = jnp.float32
I32 = jnp.int32


def _k1_body(ne_ref, ael_ref, aeh_ref, arh_ref, arr_ref, be_ref, br_ref,
             pd_ref, ps_ref):
    x = ne_ref[...]
    pd_ref[:, :D] = x @ ael_ref[...] + be_ref[...]
    pd_ref[:, D:] = x @ arr_ref[...]
    ps_ref[:, :D] = x @ aeh_ref[...]
    ps_ref[:, D:] = x @ arh_ref[...] + br_ref[...]


def _k2_body(pd_hbm, ps_hbm, sidx_hbm, didx_hbm, w0_hbm, w1_hbm,
             we_hbm, wr_hbm, sse_hbm, ssr_hbm,
             sidx_v, didx_v, pd_rows, ps_rows, we_blk, wr_blk,
             sse_v, ssr_v, w0_v, w1_v, sem_pd, sem_ps, sem_we, sem_wr):
    wid = lax.axis_index("s") * NC + lax.axis_index("c")
    pltpu.sync_copy(sidx_hbm.at[wid], sidx_v)
    pltpu.sync_copy(didx_hbm.at[wid], didx_v)
    pltpu.sync_copy(w0_hbm, w0_v)
    pltpu.sync_copy(w1_hbm, w1_v)

    zeros16 = jnp.zeros((L,), F32)

    def zero_seg(i, _):
        sse_v[pl.ds(i * L, L)] = zeros16
        ssr_v[pl.ds(i * L, L)] = zeros16
        return 0

    lax.fori_loop(0, N // L, zero_seg, 0)

    lanes = lax.iota(I32, L)

    def issue(b, slot):
        pltpu.async_copy(pd_hbm.at[didx_v.at[b]], pd_rows.at[slot],
                         sem_pd.at[slot])
        pltpu.async_copy(ps_hbm.at[sidx_v.at[b]], ps_rows.at[slot],
                         sem_ps.at[slot])

    issue(0, 0)

    def block(b, _):
        slot = b & 1
        pltpu.make_async_copy(pd_hbm.at[pl.ds(0, K2_K)], pd_rows.at[slot],
                              sem_pd.at[slot]).wait()
        pltpu.make_async_copy(ps_hbm.at[pl.ds(0, K2_K)], ps_rows.at[slot],
                              sem_ps.at[slot]).wait()

        @pl.when(b + 1 < K2_NB)
        def _():
            issue(b + 1, 1 - slot)

        @pl.when(b >= 2)
        def _():
            pltpu.make_async_copy(
                we_blk.at[pl.ds(slot * K2_K, K2_K)],
                we_hbm.at[wid, pl.ds(0, K2_K)], sem_we.at[slot]).wait()
            pltpu.make_async_copy(
                wr_blk.at[pl.ds(slot * K2_K, K2_K)],
                wr_hbm.at[wid, pl.ds(0, K2_K)], sem_wr.at[slot]).wait()

        bufsplat = jnp.full((L,), slot, I32)
        for g in range(K2_K // L):
            rows16 = jnp.full((L,), g * L, I32) + lanes

            def feat8(fi, accs):
                acc_e, acc_r = accs
                for j in range(8):
                    col_a = (jnp.full((L,), fi * 8 + j, I32) + lanes) & (
                        D - 1)
                    va = plsc.load_gather(pd_rows, [bufsplat, rows16, col_a])
                    vb = plsc.load_gather(ps_rows, [bufsplat, rows16, col_a])
                    m = jnp.exp((va + vb) * -2.0)
                    t = (1.0 - m) / (1.0 + m)
                    acc_e = acc_e + t * plsc.load_gather(w0_v, [col_a])
                    col_b = col_a + D
                    vc = plsc.load_gather(pd_rows, [bufsplat, rows16, col_b])
                    vd = plsc.load_gather(ps_rows, [bufsplat, rows16, col_b])
                    m2 = jnp.exp((vc + vd) * -2.0)
                    t2 = (1.0 - m2) / (1.0 + m2)
                    acc_r = acc_r + t2 * plsc.load_gather(w1_v, [col_a])
                return acc_e, acc_r

            acc_e, acc_r = lax.fori_loop(0, D // 8, feat8,
                                         (zeros16, zeros16))
            we16 = jnp.exp(acc_e)
            wr16 = jnp.exp(acc_r)
            off = slot * K2_K + g * L
            we_blk[pl.ds(off, L)] = we16
            wr_blk[pl.ds(off, L)] = wr16
            bsplat = jnp.full((L,), b, I32)
            s16 = plsc.load_gather(sidx_v, [bsplat, rows16])
            d16 = plsc.load_gather(didx_v, [bsplat, rows16])
            plsc.addupdate_scatter(sse_v, [s16], we16)
            plsc.addupdate_scatter(ssr_v, [d16], wr16)
        pltpu.async_copy(we_blk.at[pl.ds(slot * K2_K, K2_K)],
                         we_hbm.at[wid, pl.ds(b * K2_K, K2_K)],
                         sem_we.at[slot])
        pltpu.async_copy(wr_blk.at[pl.ds(slot * K2_K, K2_K)],
                         wr_hbm.at[wid, pl.ds(b * K2_K, K2_K)],
                         sem_wr.at[slot])
        return 0

    lax.fori_loop(0, K2_NB, block, 0)

    for slot in (0, 1):
        pltpu.make_async_copy(we_blk.at[pl.ds(slot * K2_K, K2_K)],
                              we_hbm.at[wid, pl.ds(0, K2_K)],
                              sem_we.at[slot]).wait()
        pltpu.make_async_copy(wr_blk.at[pl.ds(slot * K2_K, K2_K)],
                              wr_hbm.at[wid, pl.ds(0, K2_K)],
                              sem_wr.at[slot]).wait()
    pltpu.sync_copy(sse_v, sse_hbm.at[wid])
    pltpu.sync_copy(ssr_v, ssr_hbm.at[wid])


def _k3a_body(node_hbm, gidx_hbm, sidx_hbm, w_hbm, zeros_hbm,
              agg_hbm,
              gidx_v, sidx_v, w_v, rows_v, acc_sh, sem_a):
    cid = lax.axis_index("c")
    sid = lax.axis_index("s")
    pltpu.sync_copy(zeros_hbm, acc_sh.at[pl.ds(sid * ROWS_PT, ROWS_PT)])
    plsc.subcore_barrier()

    def chunk(c, _):
        pltpu.sync_copy(gidx_hbm.at[cid, sid, pl.ds(c * K3A_CH, K3A_CH)],
                        gidx_v)
        pltpu.sync_copy(sidx_hbm.at[cid, sid, pl.ds(c * K3A_CH, K3A_CH)],
                        sidx_v)
        pltpu.sync_copy(w_hbm.at[cid, sid, pl.ds(c * K3A_CH * K3A_K,
                                                 K3A_CH * K3A_K)], w_v)
        pltpu.async_copy(node_hbm.at[gidx_v.at[0]], rows_v.at[0],
                         sem_a.at[0])

        def block(b, _):
            slot = b & 1
            pltpu.make_async_copy(node_hbm.at[pl.ds(0, K3A_K)],
                                  rows_v.at[slot], sem_a.at[slot]).wait()

            @pl.when(b + 1 < K3A_CH)
            def _():
                pltpu.async_copy(node_hbm.at[gidx_v.at[b + 1]],
                                 rows_v.at[1 - slot], sem_a.at[1 - slot])

            rows = rows_v.at[slot]
            for e in range(K3A_K):
                wb = plsc.load_gather(
                    w_v, [jnp.full((L,), b * K3A_K + e, I32)])
                for c in range(D // L):
                    rows[e, c * L:(c + 1) * L] = (
                        rows[e, c * L:(c + 1) * L] * wb)
            pltpu.sync_copy(rows, acc_sh.at[sidx_v.at[b]], add=True)
            return 0

        lax.fori_loop(0, K3A_CH, block, 0)
        return 0

    lax.fori_loop(0, K3A_NCH, chunk, 0)
    plsc.subcore_barrier()
    pltpu.sync_copy(acc_sh.at[pl.ds(sid * ROWS_PT, ROWS_PT)],
                    agg_hbm.at[cid, sid])


def _k3b_body(node_hbm, gidx_hbm, sidx_hbm, zeros_hbm,
              msum_hbm, cnt_hbm,
              gidx_v, sidx_v, rows_v, cnt_v, acc_sh, sem_a):
    cid = lax.axis_index("c")
    sid = lax.axis_index("s")
    pltpu.sync_copy(gidx_hbm.at[cid, sid], gidx_v)
    pltpu.sync_copy(sidx_hbm.at[cid, sid], sidx_v)
    pltpu.sync_copy(zeros_hbm, acc_sh.at[pl.ds(sid * ROWS_PT, ROWS_PT)])

    zeros16 = jnp.zeros((L,), F32)

    def zero_cnt(i, _):
        cnt_v[pl.ds(i * L, L)] = zeros16
        return 0

    lax.fori_loop(0, N // L, zero_cnt, 0)
    plsc.subcore_barrier()

    lanes = lax.iota(I32, L)
    ones16 = jnp.ones((L,), F32)

    pltpu.async_copy(node_hbm.at[gidx_v.at[0]], rows_v.at[0], sem_a.at[0])

    def block(b, _):
        slot = b & 1
        pltpu.make_async_copy(node_hbm.at[pl.ds(0, K3B_K)],
                              rows_v.at[slot], sem_a.at[slot]).wait()

        @pl.when(b + 1 < K3B_NB)
        def _():
            pltpu.async_copy(node_hbm.at[gidx_v.at[b + 1]],
                             rows_v.at[1 - slot], sem_a.at[1 - slot])

        bsplat = jnp.full((L,), b, I32)
        for g in range(K3B_K // L):
            cols = jnp.full((L,), g * L, I32) + lanes
            s16 = plsc.load_gather(sidx_v, [bsplat, cols])
            plsc.addupdate_scatter(cnt_v, [s16], ones16)
        pltpu.sync_copy(rows_v.at[slot], acc_sh.at[sidx_v.at[b]], add=True)
        return 0

    lax.fori_loop(0, K3B_NB, block, 0)
    plsc.subcore_barrier()
    pltpu.sync_copy(acc_sh.at[pl.ds(sid * ROWS_PT, ROWS_PT)],
                    msum_hbm.at[cid, sid])
    pltpu.sync_copy(cnt_v, cnt_hbm.at[cid, sid])


def _k4_body(ne_ref, agge_ref, aggr_ref, m0_ref, m1_ref, sse_ref, ssr_ref,
             cnt_ref, w1_ref, b1_ref, w2_ref, b2_ref, w3_ref, b3_ref,
             out_ref):
    inv_e = 1.0 / (jnp.sum(sse_ref[...], axis=1) + 1e-9)
    inv_r = 1.0 / (jnp.sum(ssr_ref[...], axis=1) + 1e-9)
    attn = agge_ref[...] * inv_e[:, None] + aggr_ref[...] * inv_r[:, None]
    cnt = jnp.maximum(jnp.sum(cnt_ref[...], axis=1), 1.0)
    mean = (m0_ref[...] + m1_ref[...]) / cnt[:, None]
    out_ref[...] = (jnp.tanh(ne_ref[...] @ w1_ref[...] + b1_ref[...])
                    + jnp.tanh(attn @ w2_ref[...] + b2_ref[...])
                    + jnp.tanh(mean @ w3_ref[...] + b3_ref[...]))


def _sc_mesh():
    return plsc.VectorSubcoreMesh(core_axis_name="c", subcore_axis_name="s",
                                  num_cores=NC, num_subcores=NS)


_SC_PARAMS = pltpu.CompilerParams(use_tc_tiling_on_sc=False,
                                  needs_layout_passes=False)


def kernel(node_emb, er_src, er_dst, ee_src, ee_dst, rr_src, rr_dst,
           W_attn_e_w, W_attn_e_b, w0_w, w0_b,
           W_attn_r_w, W_attn_r_b, w1_w, w1_b,
           W1_w, W1_b, W2_w, W2_b, W3_w, W3_b):
    er_src = er_src.astype(I32)
    er_dst = er_dst.astype(I32)
    ee_src = ee_src.astype(I32)
    ee_dst = ee_dst.astype(I32)
    rr_src = rr_src.astype(I32)
    rr_dst = rr_dst.astype(I32)

    n_blk = 1000
    row_spec = pl.BlockSpec((n_blk, D), lambda i: (i, 0))
    wide_spec = pl.BlockSpec((n_blk, 2 * D), lambda i: (i, 0))
    w_spec = pl.BlockSpec((D, D), lambda i: (0, 0))
    b_spec = pl.BlockSpec((D,), lambda i: (0,))
    pd, ps = pl.pallas_call(
        _k1_body,
        grid=(N // n_blk,),
        in_specs=[row_spec, w_spec, w_spec, w_spec, w_spec, b_spec, b_spec],
        out_specs=[wide_spec, wide_spec],
        out_shape=[jax.ShapeDtypeStruct((N, 2 * D), F32),
                   jax.ShapeDtypeStruct((N, 2 * D), F32)],
    )(node_emb,
      W_attn_e_w[:, :D].T, W_attn_e_w[:, D:].T,
      W_attn_r_w[:, :D].T, W_attn_r_w[:, D:].T,
      W_attn_e_b, W_attn_r_b)

    k2 = pl.kernel(
        _k2_body,
        out_type=[jax.ShapeDtypeStruct((NW, K2_EPT), F32),
                  jax.ShapeDtypeStruct((NW, K2_EPT), F32),
                  jax.ShapeDtypeStruct((NW, N), F32),
                  jax.ShapeDtypeStruct((NW, N), F32)],
        mesh=_sc_mesh(),
        compiler_params=_SC_PARAMS,
        scratch_types=[pltpu.VMEM((K2_NB, K2_K), I32),
                       pltpu.VMEM((K2_NB, K2_K), I32),
                       pltpu.VMEM((2, K2_K, 2 * D), F32),
                       pltpu.VMEM((2, K2_K, 2 * D), F32),
                       pltpu.VMEM((2 * K2_K,), F32),
                       pltpu.VMEM((2 * K2_K,), F32),
                       pltpu.VMEM((N,), F32),
                       pltpu.VMEM((N,), F32),
                       pltpu.VMEM((D,), F32),
                       pltpu.VMEM((D,), F32),
                       pltpu.SemaphoreType.DMA((2,)),
                       pltpu.SemaphoreType.DMA((2,)),
                       pltpu.SemaphoreType.DMA((2,)),
                       pltpu.SemaphoreType.DMA((2,))],
    )
    we, wr, sse_p, ssr_p = k2(pd, ps,
                              er_src.reshape(NW, K2_NB, K2_K),
                              er_dst.reshape(NW, K2_NB, K2_K),
                              w0_w[0], w1_w[0])

    zeros_init = jnp.zeros((ROWS_PT, D), F32)

    k3a = pl.kernel(
        _k3a_body,
        out_type=[jax.ShapeDtypeStruct((NC, NS, ROWS_PT, D), F32)],
        mesh=_sc_mesh(),
        compiler_params=_SC_PARAMS,
        scratch_types=[pltpu.VMEM((K3A_CH, K3A_K), I32),
                       pltpu.VMEM((K3A_CH, K3A_K), I32),
                       pltpu.VMEM((K3A_CH * K3A_K,), F32),
                       pltpu.VMEM((2, K3A_K, D), F32),
                       pltpu.VMEM_SHARED((N, D), F32),
                       pltpu.SemaphoreType.DMA((2,))],
    )
    gidx_a = jnp.stack([er_dst, er_src]).reshape(NC, NS, K3A_NB, K3A_K)
    sidx_a = jnp.stack([er_src, er_dst]).reshape(NC, NS, K3A_NB, K3A_K)
    w_all = jnp.stack([we.reshape(-1), wr.reshape(-1)]).reshape(
        NC, NS, K3A_EPT)
    (agg,) = k3a(node_emb, gidx_a, sidx_a, w_all, zeros_init)

    k3b = pl.kernel(
        _k3b_body,
        out_type=[jax.ShapeDtypeStruct((NC, NS, ROWS_PT, D), F32),
                  jax.ShapeDtypeStruct((NC, NS, N), F32)],
        mesh=_sc_mesh(),
        compiler_params=_SC_PARAMS,
        scratch_types=[pltpu.VMEM((K3B_NB, K3B_K), I32),
                       pltpu.VMEM((K3B_NB, K3B_K), I32),
                       pltpu.VMEM((2, K3B_K, D), F32),
                       pltpu.VMEM((N,), F32),
                       pltpu.VMEM_SHARED((N, D), F32),
                       pltpu.SemaphoreType.DMA((2,))],
    )
    gidx_b = jnp.stack([ee_dst, rr_dst]).reshape(NC, NS, K3B_NB, K3B_K)
    sidx_b = jnp.stack([ee_src, rr_src]).reshape(NC, NS, K3B_NB, K3B_K)
    msum, cnt_p = k3b(node_emb, gidx_b, sidx_b, zeros_init)

    agg = agg.reshape(NC, N, D)
    msum = msum.reshape(NC, N, D)
    part_spec = pl.BlockSpec((n_blk, NW), lambda i: (i, 0))
    out = pl.pallas_call(
        _k4_body,
        grid=(N // n_blk,),
        in_specs=[row_spec, row_spec, row_spec, row_spec, row_spec,
                  part_spec, part_spec, part_spec,
                  w_spec, b_spec, w_spec, b_spec, w_spec, b_spec],
        out_specs=row_spec,
        out_shape=jax.ShapeDtypeStruct((N, D), F32),
    )(node_emb, agg[0], agg[1], msum[0], msum[1],
      sse_p.T, ssr_p.T, cnt_p.reshape(NW, N).T,
      W1_w.T, W1_b, W2_w.T, W2_b, W3_w.T, W3_b)
    return out
